# Initial kernel scaffold; baseline (speedup 1.0000x reference)
import jax
import jax.numpy as jnp
from jax.experimental import pallas as pl


def kernel(x, W1, g1, b1, W2, g2, b2, W3, g3, b3, W4, g4, b4, W5, g5, b5, Wd5, gd5, bd5, Wd4, gd4, bd4, Wout):
    raise NotImplementedError("write your pallas kernel here")


# speedup vs baseline: 4.6842x; 4.6842x over previous
import functools

import jax
import jax.numpy as jnp
from jax import lax
from jax.experimental import pallas as pl
from jax.experimental.pallas import tpu as pltpu
from jax.experimental.pallas import tpu_sc as plsc

B, N, K = 4, 2048, 20
M = B * N
HI = lax.Precision.HIGHEST
F32 = jnp.float32


def _dot(a, b):
    return lax.dot_general(a, b, (((1,), (0,)), ((), ())),
                           preferred_element_type=F32)


def _xx(z):
    C = z.shape[1]
    Rb = 512

    def kern(z_ref, xx_ref):
        zb = z_ref[...]
        xx_ref[...] = jnp.sum(zb * zb, axis=1, keepdims=True)

    return pl.pallas_call(
        kern,
        grid=(M // Rb,),
        in_specs=[pl.BlockSpec((Rb, C), lambda i: (i, 0))],
        out_specs=pl.BlockSpec((Rb, 1), lambda i: (i, 0)),
        out_shape=jax.ShapeDtypeStruct((M, 1), F32),
    )(z)


def _topk_idx(z, xx):
    C = z.shape[1]
    Rb = 256
    NB = N // Rb
    z3 = z.reshape(B, N, C)
    xxk = xx.reshape(B, 1, N)

    def kern(zq_ref, zk_ref, xxq_ref, xxk_ref, idx_ref):
        b = pl.program_id(0)
        zq = zq_ref[0]
        zk = zk_ref[0]
        G = lax.dot_general(zq, zk, (((1,), (1,)), ((), ())),
                            preferred_element_type=F32)
        inner = -2.0 * G
        pd = (-xxk_ref[0]) - inner - xxq_ref[...]
        iota = lax.broadcasted_iota(jnp.int32, (Rb, N), 1)
        bigi = jnp.int32(2 ** 30)
        off = b * N
        for t in range(K):
            m = jnp.max(pd, axis=1, keepdims=True)
            j = jnp.min(jnp.where(pd == m, iota, bigi), axis=1, keepdims=True)
            idx_ref[:, t:t + 1] = j + off
            pd = jnp.where(iota == j, -jnp.inf, pd)

    return pl.pallas_call(
        kern,
        grid=(B, NB),
        in_specs=[
            pl.BlockSpec((1, Rb, C), lambda b, r: (b, r, 0)),
            pl.BlockSpec((1, N, C), lambda b, r: (b, 0, 0)),
            pl.BlockSpec((Rb, 1), lambda b, r: (b * NB + r, 0)),
            pl.BlockSpec((1, 1, N), lambda b, r: (b, 0, 0)),
        ],
        out_specs=pl.BlockSpec((Rb, K), lambda b, r: (b * NB + r, 0)),
        out_shape=jax.ShapeDtypeStruct((M, K), jnp.int32),
    )(z3, z3, xx, xxk)


def _gather_diff(z, idx_flat, C):
    NW = 32
    PTS = M // NW
    Gp = 4
    GK = Gp * K
    NG = PTS // Gp
    CH = C // 16
    mesh = plsc.VectorSubcoreMesh(core_axis_name="c", subcore_axis_name="s")

    @functools.partial(
        pl.kernel,
        out_type=jax.ShapeDtypeStruct((K * M, C), F32),
        mesh=mesh,
        scratch_types=[
            pltpu.VMEM((PTS * K,), jnp.int32),
            pltpu.VMEM((GK, C), F32),
            pltpu.VMEM((K, Gp, C), F32),
            pltpu.VMEM((Gp, C), F32),
            pltpu.SemaphoreType.DMA,
            pltpu.SemaphoreType.DMA,
        ],
    )
    def k(z_hbm, idx_hbm, d_hbm, idx_v, rows_v, out_v, cent_v, sem, sem2):
        wid = lax.axis_index("s") * 2 + lax.axis_index("c")
        base_pt = wid * PTS
        pltpu.sync_copy(idx_hbm.at[pl.ds(base_pt * K, PTS * K)], idx_v)

        def group(g, carry):
            row0 = base_pt + g * Gp
            cp = pltpu.async_copy(z_hbm.at[idx_v.at[pl.ds(g * GK, GK)]],
                                  rows_v, sem)
            pltpu.sync_copy(z_hbm.at[pl.ds(row0, Gp)], cent_v)
            cp.wait()
            for pt in range(Gp):
                def chunk(ch, c2):
                    sl = pl.ds(ch * 16, 16)
                    cvec = cent_v[pt, sl]
                    for r in range(K):
                        out_v[r, pt, sl] = rows_v[pt * K + r, sl] - cvec
                    return c2
                lax.fori_loop(0, CH, chunk, 0)
            cps = [pltpu.async_copy(out_v.at[r],
                                    d_hbm.at[pl.ds(r * M + row0, Gp)], sem2)
                   for r in range(K)]
            for c in cps:
                c.wait()
            return carry

        lax.fori_loop(0, NG, group, 0)

    return k(z, idx_flat)


def _mm_segstats(diffs, z, Wst, Cr, O):
    Rb = 512
    NI = M // Rb
    C = z.shape[1]

    def kern(d_ref, z_ref, w_ref, mx_ref, sm_ref, sq_ref):
        j = pl.program_id(1)
        f = jnp.concatenate([d_ref[:, :Cr], z_ref[:, :Cr]], axis=1)
        t = _dot(f, w_ref[...])

        @pl.when(j == 0)
        def _():
            mx_ref[...] = t
            sm_ref[...] = t
            sq_ref[...] = t * t

        @pl.when(j > 0)
        def _():
            mx_ref[...] = jnp.maximum(mx_ref[...], t)
            sm_ref[...] += t
            sq_ref[...] += t * t

    return pl.pallas_call(
        kern,
        grid=(NI, K),
        in_specs=[
            pl.BlockSpec((Rb, C), lambda i, j: (j * NI + i, 0)),
            pl.BlockSpec((Rb, C), lambda i, j: (i, 0)),
            pl.BlockSpec((2 * Cr, O), lambda i, j: (0, 0)),
        ],
        out_specs=[pl.BlockSpec((Rb, O), lambda i, j: (i, 0))] * 3,
        out_shape=[jax.ShapeDtypeStruct((M, O), F32)] * 3,
    )(diffs, z, Wst)


def _bn_stats(ysum, ysq, O):
    Rb = 512

    def kern(s_ref, q_ref, S_ref):
        i = pl.program_id(0)

        @pl.when(i == 0)
        def _():
            S_ref[...] = jnp.zeros_like(S_ref)

        S_ref[0:1, :] += jnp.sum(s_ref[...], axis=0, keepdims=True)
        S_ref[1:2, :] += jnp.sum(q_ref[...], axis=0, keepdims=True)

    return pl.pallas_call(
        kern,
        grid=(M // Rb,),
        in_specs=[pl.BlockSpec((Rb, O), lambda i: (i, 0))] * 2,
        out_specs=pl.BlockSpec((8, O), lambda i: (0, 0)),
        out_shape=jax.ShapeDtypeStruct((8, O), F32),
    )(ysum, ysq)


def _bn_norm_max(ymax, S, g, b, O):
    Rb = 512
    cnt = float(M * K)

    def kern(mx_ref, S_ref, g_ref, b_ref, o_ref):
        mean = S_ref[0:1, :] / cnt
        var = S_ref[1:2, :] / cnt - mean * mean
        t = (g_ref[...] * (mx_ref[...] - mean)
             / jnp.sqrt(var + 1e-5) + b_ref[...])
        o_ref[...] = jnp.maximum(t, 0.2 * t)

    return pl.pallas_call(
        kern,
        grid=(M // Rb,),
        in_specs=[
            pl.BlockSpec((Rb, O), lambda i: (i, 0)),
            pl.BlockSpec((8, O), lambda i: (0, 0)),
            pl.BlockSpec((1, O), lambda i: (0, 0)),
            pl.BlockSpec((1, O), lambda i: (0, 0)),
        ],
        out_specs=pl.BlockSpec((Rb, O), lambda i: (i, 0)),
        out_shape=jax.ShapeDtypeStruct((M, O), F32),
    )(ymax, S, g, b)


def _edge_layer(z, Wst, g, b):
    O = Wst.shape[1]
    Cr = Wst.shape[0] // 2
    xx = _xx(z)
    idx = _topk_idx(z, xx)
    diffs = _gather_diff(z, idx.reshape(-1), z.shape[1])
    mx, sm, sq = _mm_segstats(diffs, z, Wst, Cr, O)
    S = _bn_stats(sm, sq, O)
    return _bn_norm_max(mx, S, g.reshape(1, -1), b.reshape(1, -1), O)


def _padc(w, t):
    return jnp.pad(w, ((0, 0), (0, t - w.shape[1])))


def _padr(w, t):
    return jnp.pad(w, ((0, t - w.shape[0]), (0, 0)))


def _pad1(v, t, val):
    return jnp.pad(v, (0, t - v.shape[0]), constant_values=val)


def _head_mm_stats(xs, Ws, O):
    Rb = 512
    nin = len(xs)

    def kern(*refs):
        i = pl.program_id(0)
        y = _dot(refs[0][...], refs[nin][...])
        for j in range(1, nin):
            y = y + _dot(refs[j][...], refs[nin + j][...])
        y_ref, S_ref = refs[2 * nin], refs[2 * nin + 1]
        y_ref[...] = y

        @pl.when(i == 0)
        def _():
            S_ref[...] = jnp.zeros_like(S_ref)

        S_ref[0:1, :] += jnp.sum(y, axis=0, keepdims=True)
        S_ref[1:2, :] += jnp.sum(y * y, axis=0, keepdims=True)

    in_specs = [pl.BlockSpec((Rb, x.shape[1]), lambda i: (i, 0)) for x in xs]
    in_specs += [pl.BlockSpec(w.shape, lambda i: (0, 0)) for w in Ws]
    return pl.pallas_call(
        kern,
        grid=(M // Rb,),
        in_specs=in_specs,
        out_specs=[
            pl.BlockSpec((Rb, O), lambda i: (i, 0)),
            pl.BlockSpec((8, O), lambda i: (0, 0)),
        ],
        out_shape=[
            jax.ShapeDtypeStruct((M, O), F32),
            jax.ShapeDtypeStruct((8, O), F32),
        ],
    )(*xs, *Ws)


def _bn_then_mm(y, S, g, b, Wn, On, with_stats):
    Rb = 512
    Cy = y.shape[1]
    cnt = float(M)

    def kern(y_ref, S_ref, g_ref, b_ref, w_ref, o_ref, *maybe_S):
        i = pl.program_id(0)
        mean = S_ref[0:1, :] / cnt
        var = S_ref[1:2, :] / cnt - mean * mean
        scale = g_ref[...] * lax.rsqrt(var + 1e-5)
        t = (y_ref[...] - mean) * scale + b_ref[...]
        z = jnp.maximum(t, 0.2 * t)
        o = _dot(z, w_ref[...])
        o_ref[...] = o
        if with_stats:
            S2_ref = maybe_S[0]

            @pl.when(i == 0)
            def _():
                S2_ref[...] = jnp.zeros_like(S2_ref)

            S2_ref[0:1, :] += jnp.sum(o, axis=0, keepdims=True)
            S2_ref[1:2, :] += jnp.sum(o * o, axis=0, keepdims=True)

    out_specs = [pl.BlockSpec((Rb, On), lambda i: (i, 0))]
    out_shape = [jax.ShapeDtypeStruct((M, On), F32)]
    if with_stats:
        out_specs.append(pl.BlockSpec((8, On), lambda i: (0, 0)))
        out_shape.append(jax.ShapeDtypeStruct((8, On), F32))
    res = pl.pallas_call(
        kern,
        grid=(M // Rb,),
        in_specs=[
            pl.BlockSpec((Rb, Cy), lambda i: (i, 0)),
            pl.BlockSpec((8, Cy), lambda i: (0, 0)),
            pl.BlockSpec((1, Cy), lambda i: (0, 0)),
            pl.BlockSpec((1, Cy), lambda i: (0, 0)),
            pl.BlockSpec((Cy, On), lambda i: (0, 0)),
        ],
        out_specs=out_specs,
        out_shape=out_shape,
    )(y, S, g.reshape(1, -1), b.reshape(1, -1), Wn)
    return res if with_stats else (res[0], None)


def kernel(x, W1, g1, b1, W2, g2, b2, W3, g3, b3, W4, g4, b4,
           W5, g5, b5, Wd5, gd5, bd5, Wd4, gd4, bd4, Wout):
    z = jnp.transpose(x, (0, 2, 1)).reshape(M, 15)
    z = jnp.pad(z, ((0, 0), (0, 113)))

    def prep(W, C, Ot):
        return _padc(W, Ot)

    Wst1 = prep(W1, 15, 128)
    Wst2 = prep(W2, 64, 128)
    Wst3 = prep(W3, 64, 128)
    Wst4 = prep(W4, 128, 256)

    x1 = _edge_layer(z, Wst1, _pad1(g1, 128, 1.0), _pad1(b1, 128, 0.0))
    x2 = _edge_layer(x1, Wst2, _pad1(g2, 128, 1.0), _pad1(b2, 128, 0.0))
    x3 = _edge_layer(x2, Wst3, g3, b3)
    x4 = _edge_layer(x3, Wst4, g4, b4)

    y5, S5 = _head_mm_stats(
        [x1, x2, x3, x4],
        [_padr(W5[0:64], 128), _padr(W5[64:128], 128),
         W5[128:256], W5[256:512]],
        256)
    y6, S6 = _bn_then_mm(y5, S5, g5, b5, Wd5, 128, True)
    y7, S7 = _bn_then_mm(y6, S6, gd5, bd5, Wd4, 64, True)
    o, _ = _bn_then_mm(y7, S7, gd4, bd4, Wout, 2, False)
    return jnp.transpose(o.reshape(B, N, 2), (0, 2, 1))
